# Initial kernel scaffold; baseline (speedup 1.0000x reference)
import jax
import jax.numpy as jnp
from jax.experimental import pallas as pl


def kernel(queries, keys):
    raise NotImplementedError("write your pallas kernel here")


# speedup vs baseline: 4.0126x; 4.0126x over previous
import functools

import jax
import jax.numpy as jnp
from jax import lax
from jax.experimental import pallas as pl
from jax.experimental.pallas import tpu as pltpu

NN = 75
GRP = 128
KB = 16384
SUB = 2048
RB = 8
PAD_VAL = 1e18
BIG_F = 3.0e38
BIG_I = 2**31 - 1


def _dist_groupmin_body(qt_ref, kt_ref, out_ref):
    q = qt_ref[...]
    qsq = jnp.sum(q * q, axis=1, keepdims=True)
    for s in range(KB // SUB):
        kt = kt_ref[:, s * SUB:(s + 1) * SUB]
        ksq = jnp.sum(kt * kt, axis=0, keepdims=True)
        d = qsq - 2.0 * jnp.dot(q, kt, preferred_element_type=jnp.float32) + ksq
        ng = SUB // GRP
        dg = d.reshape(d.shape[0], ng, GRP)
        out_ref[:, s * ng:(s + 1) * ng] = jnp.min(dg, axis=2)


def _top_groups_body(gm_ref, out_ref):
    d = gm_ref[...]
    rq, g = d.shape
    ii = lax.broadcasted_iota(jnp.int32, (rq, g), 1)
    li = lax.broadcasted_iota(jnp.int32, (rq, 128), 1)
    out0 = jnp.zeros((rq, 128), jnp.int32)

    def step(t, carry):
        d, am = carry
        m = jnp.min(d, axis=1, keepdims=True)
        lane = jnp.min(jnp.where(d == m, ii, BIG_I), axis=1, keepdims=True)
        am = jnp.where(li == t, lane, am)
        d = jnp.where(ii == lane, BIG_F, d)
        return d, am

    _, am = lax.fori_loop(0, NN, step, (d, out0))
    out_ref[...] = am


def _rescore1_body(sref, q_ref, k_ref, d_ref):
    kb = k_ref[0]
    q = q_ref[0]
    qsq = jnp.sum(q * q)
    ksq = jnp.sum(kb * kb, axis=0, keepdims=True)
    dot = jnp.dot(q, kb, preferred_element_type=jnp.float32)
    d_ref[0] = qsq - 2.0 * dot + ksq


def _rescore_body(sref, qt_ref, *refs):
    kr = refs[:RB]
    dall_ref, iall_ref = refs[RB], refs[RB + 1]
    r = pl.program_id(0)
    j = pl.program_id(1)
    drows, irows = [], []
    for i in range(RB):
        kb = kr[i][0]
        qc = qt_ref[:, i:i + 1]
        qsq = jnp.sum(qc * qc)
        ksq = jnp.sum(kb * kb, axis=0, keepdims=True)
        dot = jnp.sum(kb * qc, axis=0, keepdims=True)
        drows.append(qsq - 2.0 * dot + ksq)
        sid = sref[(r * RB + i) * NN + j]
        irows.append(sid * GRP + lax.broadcasted_iota(jnp.int32, (1, GRP), 1))
    dall_ref[...] = jnp.concatenate(drows, axis=0)
    iall_ref[...] = jnp.concatenate(irows, axis=0)


def _final_topk_body(d_ref, i_ref, dt_ref, it_ref):
    d = d_ref[...]
    gi = i_ref[...]
    rq, c = d.shape
    ii = lax.broadcasted_iota(jnp.int32, (rq, c), 1)
    li = lax.broadcasted_iota(jnp.int32, (rq, 128), 1)
    d0 = jnp.zeros((rq, 128), jnp.float32)
    i0 = jnp.zeros((rq, 128), jnp.int32)

    def step(t, carry):
        d, dt, it = carry
        m = jnp.min(d, axis=1, keepdims=True)
        hit = d == m
        cidx = jnp.min(jnp.where(hit, gi, BIG_I), axis=1, keepdims=True)
        lane = jnp.min(jnp.where(hit & (gi == cidx), ii, BIG_I),
                       axis=1, keepdims=True)
        dt = jnp.where(li == t, m, dt)
        it = jnp.where(li == t, cidx, it)
        d = jnp.where(ii == lane, BIG_F, d)
        return d, dt, it

    _, dt, it = lax.fori_loop(0, NN, step, (d, d0, i0))
    dt_ref[...] = dt
    it_ref[...] = it


def kernel(queries, keys):
    qn, dim = queries.shape
    kn = keys.shape[0]
    nb = (kn + KB - 1) // KB
    kpad = nb * KB
    g = kpad // GRP
    keys_p = jnp.pad(keys, ((0, kpad - kn), (0, 0)), constant_values=PAD_VAL)
    keys_t = keys_p.T
    keys_g = jnp.swapaxes(keys_p.reshape(g, GRP, dim), 1, 2)
    qt3 = jnp.swapaxes(queries.reshape(qn // RB, RB, dim), 1, 2)

    gmins = pl.pallas_call(
        _dist_groupmin_body,
        grid=(nb,),
        in_specs=[
            pl.BlockSpec((qn, dim), lambda b: (0, 0)),
            pl.BlockSpec((dim, KB), lambda b: (0, b)),
        ],
        out_specs=pl.BlockSpec((qn, KB // GRP), lambda b: (0, b)),
        out_shape=jax.ShapeDtypeStruct((qn, g), jnp.float32),
    )(queries, keys_t)

    rq_b = qn // 8
    sids_b = pl.pallas_call(
        _top_groups_body,
        grid=(8,),
        in_specs=[pl.BlockSpec((rq_b, g), lambda b: (b, 0))],
        out_specs=pl.BlockSpec((rq_b, 128), lambda b: (b, 0)),
        out_shape=jax.ShapeDtypeStruct((qn, 128), jnp.int32),
    )(gmins)
    sids_x = sids_b[:, :NN]
    sflat_x = sids_x.reshape(-1)
    keys_g_x = jnp.swapaxes(keys_p.reshape(g, GRP, dim), 1, 2)
    q3_x = queries.reshape(qn, 1, dim)
    cw_x = NN * GRP
    dx3 = pl.pallas_call(
        _rescore1_body,
        grid_spec=pltpu.PrefetchScalarGridSpec(
            num_scalar_prefetch=1,
            grid=(qn, NN),
            in_specs=[
                pl.BlockSpec((1, 1, dim), lambda q, j, s: (q, 0, 0)),
                pl.BlockSpec((1, dim, GRP),
                             lambda q, j, s: (s[q * NN + j], 0, 0)),
            ],
            out_specs=pl.BlockSpec((1, 1, GRP), lambda q, j, s: (q, 0, j)),
        ),
        out_shape=jax.ShapeDtypeStruct((qn, 1, cw_x), jnp.float32),
    )(sflat_x, q3_x, keys_g_x)
    dx = dx3.reshape(qn, cw_x)
    ix = (sids_x[:, :, None] * GRP
          + jnp.arange(GRP, dtype=jnp.int32)[None, None, :]).reshape(qn, cw_x)
    rq2 = qn // 8
    dt, it = pl.pallas_call(
        _final_topk_body,
        grid=(8,),
        in_specs=[
            pl.BlockSpec((rq2, cw_x), lambda b: (b, 0)),
            pl.BlockSpec((rq2, cw_x), lambda b: (b, 0)),
        ],
        out_specs=[
            pl.BlockSpec((rq2, 128), lambda b: (b, 0)),
            pl.BlockSpec((rq2, 128), lambda b: (b, 0)),
        ],
        out_shape=[
            jax.ShapeDtypeStruct((qn, 128), jnp.float32),
            jax.ShapeDtypeStruct((qn, 128), jnp.int32),
        ],
    )(dx, ix)
    return dt[:, :NN], it[:, :NN]
